# baseline (speedup 1.0000x reference)
import functools

import numpy as np
import jax
import jax.numpy as jnp
from jax import lax
from jax.experimental import pallas as pl
from jax.experimental.pallas import tpu as pltpu

_LANES = 128


def _apply_stage(x, CH, k, s, off):
    if s >= 8:
        nb = CH // (2 * s)
        x4 = x.reshape(nb, 2, s, _LANES)
        u = x4[:, 0]
        v = x4[:, 1]
        mn = jnp.minimum(u, v)
        mx = jnp.maximum(u, v)
        if k >= CH:
            asc = (off & k) == 0
            nu = jnp.where(asc, mn, mx)
            nv = jnp.where(asc, mx, mn)
        else:
            blk = lax.broadcasted_iota(jnp.int32, (nb, 1, _LANES), 0)
            pat = ((blk * (2 * s)) & k) == 0
            nu = jnp.where(pat, mn, mx)
            nv = jnp.where(pat, mx, mn)
        y = jnp.concatenate([nu[:, None], nv[:, None]], axis=1)
        return y.reshape(CH, _LANES)
    rows = lax.broadcasted_iota(jnp.int32, (CH, _LANES), 0)
    bit_clear = (rows & s) == 0
    p = jnp.where(bit_clear, jnp.roll(x, -s, axis=0), jnp.roll(x, s, axis=0))
    if k >= CH:
        asc = (off & k) == 0
    else:
        asc = (rows & k) == 0
    take_min = bit_clear == asc
    return jnp.where(take_min, jnp.minimum(x, p), jnp.maximum(x, p))


def _far_stage(scr, N, CH, k, s):
    ratio = s // CH

    def body(t, carry):
        q = t // ratio
        r = t - q * ratio
        u_off = q * (2 * s) + r * CH
        v_off = u_off + s
        u = scr[pl.ds(u_off, CH), :]
        v = scr[pl.ds(v_off, CH), :]
        mn = jnp.minimum(u, v)
        mx = jnp.maximum(u, v)
        asc = (u_off & k) == 0
        scr[pl.ds(u_off, CH), :] = jnp.where(asc, mn, mx)
        scr[pl.ds(v_off, CH), :] = jnp.where(asc, mx, mn)
        return carry

    lax.fori_loop(0, N // (2 * CH), body, 0)


def _chunk_pass(scr, N, CH, stages, first_mul=None):
    def body(t, carry):
        off = t * CH
        if first_mul is None:
            x = scr[pl.ds(off, CH), :]
        else:
            x_ref, m_ref = first_mul
            x = x_ref[0, pl.ds(off, CH), :] * m_ref[pl.ds(off, CH), :]
        for (k, s) in stages:
            x = _apply_stage(x, CH, k, s, off)
        scr[pl.ds(off, CH), :] = x
        return carry

    lax.fori_loop(0, N // CH, body, 0)


def _sort_cols(scr, N, CH, first_mul):
    init = []
    k = 2
    while k <= min(CH, N):
        s = k // 2
        while s > 0:
            init.append((k, s))
            s //= 2
        k *= 2
    _chunk_pass(scr, N, CH, init, first_mul=first_mul)
    while k <= N:
        s = k // 2
        while s >= CH:
            _far_stage(scr, N, CH, k, s)
            s //= 2
        tail = []
        while s > 0:
            tail.append((k, s))
            s //= 2
        _chunk_pass(scr, N, CH, tail)
        k *= 2


def _swd_kernel(N, CH, x_ref, m_ref, out_ref, scr_cur, scr_keep):
    j = pl.program_id(0) % 2
    nch = N // CH

    _sort_cols(scr_cur, N, CH, (x_ref, m_ref))

    @pl.when(j == 0)
    def _():
        def cp_body(t, carry):
            scr_keep[pl.ds(t * CH, CH), :] = scr_cur[pl.ds(t * CH, CH), :]
            return carry

        lax.fori_loop(0, nch, cp_body, 0)

    @pl.when(j == 1)
    def _():
        def acc_body(t, acc):
            d = jnp.abs(scr_cur[pl.ds(t * CH, CH), :] -
                        scr_keep[pl.ds(t * CH, CH), :])
            return acc + jnp.sum(d, axis=0, keepdims=True)

        out_ref[0] = lax.fori_loop(0, nch, acc_body,
                                   jnp.zeros((1, _LANES), jnp.float32))


def _scale_colsums(f1, f2, um, CH=512):
    B, C, h, w = f1.shape
    stride = um.shape[1] // h
    N = h * w
    BC = B * C
    m = um[:, ::stride, ::stride].reshape(B, N)
    m_bc = jnp.repeat(m.T, C, axis=1)
    a_t = f1.reshape(BC, N).T
    b_t = f2.reshape(BC, N).T
    x = jnp.stack([a_t, b_t])
    ncb = BC // _LANES
    out = pl.pallas_call(
        functools.partial(_swd_kernel, N, CH),
        grid=(2 * ncb,),
        in_specs=[
            pl.BlockSpec((1, N, _LANES), lambda g: (g % 2, 0, g // 2)),
            pl.BlockSpec((N, _LANES), lambda g: (0, g // 2)),
        ],
        out_specs=pl.BlockSpec((1, 1, _LANES), lambda g: (g // 2, 0, 0)),
        out_shape=jax.ShapeDtypeStruct((ncb, 1, _LANES), jnp.float32),
        scratch_shapes=[
            pltpu.VMEM((N, _LANES), jnp.float32),
            pltpu.VMEM((N, _LANES), jnp.float32),
        ],
    )(x, m_bc)
    return out.reshape(BC), m, N


@jax.jit
def kernel(feat_t1_s0, feat_t1_s1, feat_t2_s0, feat_t2_s1, target_mask):
    um = (1 - target_mask).astype(jnp.float32)
    losses = []
    for f1, f2 in ((feat_t1_s0, feat_t2_s0), (feat_t1_s1, feat_t2_s1)):
        colsums, m, N = _scale_colsums(f1, f2, um)
        B, C = f1.shape[0], f1.shape[1]
        valid = jnp.maximum(jnp.sum(m, axis=1), 1.0)
        per_b = colsums.reshape(B, C).sum(axis=1) / valid
        losses.append(jnp.sum(per_b) / (B * C * N))
    return (losses[0] + losses[1]) * 0.5


# speedup vs baseline: 1.0765x; 1.0765x over previous
import functools

import numpy as np
import jax
import jax.numpy as jnp
from jax import lax
from jax.experimental import pallas as pl
from jax.experimental.pallas import tpu as pltpu

_LANES = 128


def _apply_stage(x, CH, k, s, off):
    if s >= 8:
        nb = CH // (2 * s)
        x4 = x.reshape(nb, 2, s, _LANES)
        u = x4[:, 0]
        v = x4[:, 1]
        mn = jnp.minimum(u, v)
        mx = jnp.maximum(u, v)
        if k >= CH:
            asc = (off & k) == 0
            nu = jnp.where(asc, mn, mx)
            nv = jnp.where(asc, mx, mn)
        else:
            blk = lax.broadcasted_iota(jnp.int32, (nb, 1, _LANES), 0)
            pat = ((blk * (2 * s)) & k) == 0
            nu = jnp.where(pat, mn, mx)
            nv = jnp.where(pat, mx, mn)
        y = jnp.concatenate([nu[:, None], nv[:, None]], axis=1)
        return y.reshape(CH, _LANES)
    rows = lax.broadcasted_iota(jnp.int32, (CH, _LANES), 0)
    bit_clear = (rows & s) == 0
    p = jnp.where(bit_clear, jnp.roll(x, -s, axis=0), jnp.roll(x, s, axis=0))
    if k >= CH:
        asc = (off & k) == 0
    else:
        asc = (rows & k) == 0
    take_min = bit_clear == asc
    return jnp.where(take_min, jnp.minimum(x, p), jnp.maximum(x, p))


def _far_stage(scr, N, CH, k, s):
    ratio = s // CH

    def body(t, carry):
        q = t // ratio
        r = t - q * ratio
        u_off = q * (2 * s) + r * CH
        v_off = u_off + s
        u = scr[pl.ds(u_off, CH), :]
        v = scr[pl.ds(v_off, CH), :]
        mn = jnp.minimum(u, v)
        mx = jnp.maximum(u, v)
        asc = (u_off & k) == 0
        scr[pl.ds(u_off, CH), :] = jnp.where(asc, mn, mx)
        scr[pl.ds(v_off, CH), :] = jnp.where(asc, mx, mn)
        return carry

    lax.fori_loop(0, N // (2 * CH), body, 0)


def _chunk_pass(scr, N, CH, stages, first_mul=None):
    def body(t, carry):
        off = t * CH
        if first_mul is None:
            x = scr[pl.ds(off, CH), :]
        else:
            x_ref, m_ref = first_mul
            x = x_ref[0, pl.ds(off, CH), :] * m_ref[pl.ds(off, CH), :]
        for (k, s) in stages:
            x = _apply_stage(x, CH, k, s, off)
        scr[pl.ds(off, CH), :] = x
        return carry

    lax.fori_loop(0, N // CH, body, 0)


def _sort_cols(scr, N, CH, first_mul):
    init = []
    k = 2
    while k <= min(CH, N):
        s = k // 2
        while s > 0:
            init.append((k, s))
            s //= 2
        k *= 2
    _chunk_pass(scr, N, CH, init, first_mul=first_mul)
    while k <= N:
        s = k // 2
        while s >= CH:
            _far_stage(scr, N, CH, k, s)
            s //= 2
        tail = []
        while s > 0:
            tail.append((k, s))
            s //= 2
        _chunk_pass(scr, N, CH, tail)
        k *= 2


def _swd_kernel(N, CH, x_ref, m_ref, out_ref, scr_cur, scr_keep):
    j = pl.program_id(0) % 2
    nch = N // CH

    _sort_cols(scr_cur, N, CH, (x_ref, m_ref))

    @pl.when(j == 0)
    def _():
        def cp_body(t, carry):
            scr_keep[pl.ds(t * CH, CH), :] = scr_cur[pl.ds(t * CH, CH), :]
            return carry

        lax.fori_loop(0, nch, cp_body, 0)

    @pl.when(j == 1)
    def _():
        def acc_body(t, acc):
            d = jnp.abs(scr_cur[pl.ds(t * CH, CH), :] -
                        scr_keep[pl.ds(t * CH, CH), :])
            return acc + jnp.sum(d, axis=0, keepdims=True)

        out_ref[0] = lax.fori_loop(0, nch, acc_body,
                                   jnp.zeros((1, _LANES), jnp.float32))


def _scale_colsums(f1, f2, um, CH=256):
    B, C, h, w = f1.shape
    stride = um.shape[1] // h
    N = h * w
    BC = B * C
    m = um[:, ::stride, ::stride].reshape(B, N)
    m_bc = jnp.repeat(m.T, C, axis=1)
    a_t = f1.reshape(BC, N).T
    b_t = f2.reshape(BC, N).T
    x = jnp.stack([a_t, b_t])
    ncb = BC // _LANES
    out = pl.pallas_call(
        functools.partial(_swd_kernel, N, CH),
        grid=(2 * ncb,),
        in_specs=[
            pl.BlockSpec((1, N, _LANES), lambda g: (g % 2, 0, g // 2)),
            pl.BlockSpec((N, _LANES), lambda g: (0, g // 2)),
        ],
        out_specs=pl.BlockSpec((1, 1, _LANES), lambda g: (g // 2, 0, 0)),
        out_shape=jax.ShapeDtypeStruct((ncb, 1, _LANES), jnp.float32),
        scratch_shapes=[
            pltpu.VMEM((N, _LANES), jnp.float32),
            pltpu.VMEM((N, _LANES), jnp.float32),
        ],
    )(x, m_bc)
    return out.reshape(BC), m, N


@jax.jit
def kernel(feat_t1_s0, feat_t1_s1, feat_t2_s0, feat_t2_s1, target_mask):
    um = (1 - target_mask).astype(jnp.float32)
    losses = []
    for f1, f2 in ((feat_t1_s0, feat_t2_s0), (feat_t1_s1, feat_t2_s1)):
        colsums, m, N = _scale_colsums(f1, f2, um)
        B, C = f1.shape[0], f1.shape[1]
        valid = jnp.maximum(jnp.sum(m, axis=1), 1.0)
        per_b = colsums.reshape(B, C).sum(axis=1) / valid
        losses.append(jnp.sum(per_b) / (B * C * N))
    return (losses[0] + losses[1]) * 0.5
